# Initial kernel scaffold; baseline (speedup 1.0000x reference)
import jax
import jax.numpy as jnp
from jax.experimental import pallas as pl


def kernel(x, w1, b1, w2, b2, wf, fc1_b, fc2_wT, fc2_b, fc3_wT, fc3_b):
    raise NotImplementedError("write your pallas kernel here")


# speedup vs baseline: 12.9212x; 12.9212x over previous
import jax
import jax.numpy as jnp
import numpy as np
from jax.experimental import pallas as pl
from jax.experimental.pallas import tpu as pltpu

IMG = 28
P1 = 726
CH1S = 768
PW = 704
G2 = 13
C1S = 256
Q2 = 144
M2S = 128
C1, C2 = 6, 16
FC1, FC2, FC3 = 120, 84, 10

_BF = jnp.bfloat16
_F32 = jnp.float32


def _np_sel_matrix():
    s = np.zeros((PW, C1S), np.float32)
    for r in range(G2):
        for c in range(G2):
            s[56 * r + 2 * c, G2 * r + c] = 1.0
    return s


_S_NP = _np_sel_matrix()

_QV = np.array([G2 * oh + ow for oh in range(11) for ow in range(11)])

_D2_NP = []
for _ki in range(3):
    for _kj in range(3):
        _d = np.zeros((C1S, Q2), np.float32)
        _d[_QV + G2 * _ki + _kj, _QV] = 1.0
        _D2_NP.append(_d)

_WF_SRC = np.array([640 * c + 112 * r + 4 * s
                    for c in range(C2) for r in range(5) for s in range(5)])
_WF_DST = np.array([128 * c + 26 * r + 2 * s
                    for c in range(C2) for r in range(5) for s in range(5)])


def _build_w1big(w1):
    w1r = w1.reshape(C1, 9)
    acc = None
    for t in range(9):
        ki, kj = divmod(t, 3)
        d = jnp.eye(IMG * IMG, CH1S, k=-(IMG * ki + kj), dtype=_F32)
        term = w1r[None, :, t, None] * d[:, None, :]
        acc = term if acc is None else acc + term
    return acc.reshape(IMG * IMG, C1 * CH1S).astype(_BF)


def _build_w2big(w2):
    w2r = w2.reshape(C2, C1, 3, 3)
    acc = None
    for t in range(9):
        ki, kj = divmod(t, 3)
        d = jnp.asarray(_D2_NP[t])
        term = (w2r[:, :, ki, kj].T[:, None, :, None]
                * d[None, :, None, :])
        acc = term if acc is None else acc + term
    return acc.reshape(C1 * C1S, C2 * Q2).astype(_BF)


def _lenet_body(x_ref, w1b_ref, s_ref, b1_ref, w2b_ref, b2_ref,
                wf_ref, fb1_ref, w2f_ref, fb2_ref, w3f_ref, fb3_ref,
                out_ref):
    x = x_ref[...]
    y1 = jnp.dot(x, w1b_ref[...], preferred_element_type=_F32)

    mcs = []
    sel = s_ref[...]
    for c in range(C1):
        b = c * CH1S
        m = jnp.maximum(
            jnp.maximum(y1[:, b:b + PW], y1[:, b + 1:b + 1 + PW]),
            jnp.maximum(y1[:, b + IMG:b + IMG + PW],
                        y1[:, b + IMG + 1:b + IMG + 1 + PW]))
        mcs.append(jnp.dot(m.astype(_BF), sel, preferred_element_type=_F32))
    m1c = jnp.concatenate(mcs, axis=1)
    a1 = jnp.maximum(m1c + b1_ref[...], 0.0).astype(_BF)

    z2 = jnp.dot(a1, w2b_ref[...], preferred_element_type=_F32)
    a2 = jnp.maximum(z2 + b2_ref[...], 0.0)

    m2s = []
    for o in range(C2):
        b = o * Q2
        m = jnp.maximum(
            jnp.maximum(a2[:, b:b + M2S], a2[:, b + 1:b + 1 + M2S]),
            jnp.maximum(a2[:, b + G2:b + G2 + M2S],
                        a2[:, b + G2 + 1:b + G2 + 1 + M2S]))
        m2s.append(m)
    m2 = jnp.concatenate(m2s, axis=1).astype(_BF)

    h = jnp.dot(m2, wf_ref[...], preferred_element_type=_F32)
    h = jnp.maximum(h + fb1_ref[...], 0.0)
    h = jnp.dot(h, w2f_ref[...], preferred_element_type=_F32)
    h = jnp.maximum(h + fb2_ref[...], 0.0)
    z = jnp.dot(h, w3f_ref[...], preferred_element_type=_F32) + fb3_ref[...]
    zmax = jnp.max(z, axis=-1, keepdims=True)
    lse = jnp.log(jnp.sum(jnp.exp(z - zmax), axis=-1, keepdims=True)) + zmax
    out_ref[...] = z - lse


def kernel(x, w1, b1, w2, b2, wf, fc1_b, fc2_wT, fc2_b, fc3_wT, fc3_b,
           *, block_n=256):
    n = x.shape[0]
    x_flat = x.reshape(n, -1).astype(_BF)

    npad = ((n + block_n - 1) // block_n) * block_n
    if npad != n:
        x_flat = jnp.pad(x_flat, ((0, npad - n), (0, 0)))

    w1big = _build_w1big(w1)
    w2big = _build_w2big(w2)
    smat = jnp.asarray(_S_NP).astype(_BF)
    b1big = jnp.repeat(b1, C1S).reshape(1, -1)
    b2big = jnp.repeat(b2, Q2).reshape(1, -1)
    wf2 = (jnp.zeros((C2 * M2S, FC1), _F32)
           .at[_WF_DST].set(wf[_WF_SRC]).astype(_BF))

    def resident(shape):
        return pl.BlockSpec(shape, lambda b: tuple(0 for _ in shape))

    out = pl.pallas_call(
        _lenet_body,
        out_shape=jax.ShapeDtypeStruct((npad, FC3), _F32),
        grid=(npad // block_n,),
        in_specs=[
            pl.BlockSpec((block_n, IMG * IMG), lambda b: (b, 0)),
            resident((IMG * IMG, C1 * CH1S)),
            resident((PW, C1S)),
            resident((1, C1 * C1S)),
            resident((C1 * C1S, C2 * Q2)),
            resident((1, C2 * Q2)),
            resident((C2 * M2S, FC1)),
            resident((1, FC1)),
            resident((FC1, FC2)), resident((1, FC2)),
            resident((FC2, FC3)), resident((1, FC3)),
        ],
        out_specs=pl.BlockSpec((block_n, FC3), lambda b: (b, 0)),
        compiler_params=pltpu.CompilerParams(
            dimension_semantics=("parallel",),
            vmem_limit_bytes=60 << 20,
        ),
    )(x_flat, w1big, smat, b1big, w2big, b2big, wf2,
      fc1_b, fc2_wT, fc2_b, fc3_wT, fc3_b)
    return out[:n]


# speedup vs baseline: 14.9472x; 1.1568x over previous
import jax
import jax.numpy as jnp
import numpy as np
from jax.experimental import pallas as pl
from jax.experimental.pallas import tpu as pltpu

IMG = 28
P1 = 726
CH1S = 768
PW = 704
G2 = 13
C1S = 256
Q2 = 144
M2S = 128
C1, C2 = 6, 16
FC1, FC2, FC3 = 120, 84, 10

_BF = jnp.bfloat16
_F32 = jnp.float32


def _np_sel_matrix():
    s = np.zeros((PW, C1S), np.float32)
    for r in range(G2):
        for c in range(G2):
            s[56 * r + 2 * c, G2 * r + c] = 1.0
    return s


_S_NP = _np_sel_matrix()

_QV = np.array([G2 * oh + ow for oh in range(11) for ow in range(11)])

_D2_NP = []
for _ki in range(3):
    for _kj in range(3):
        _d = np.zeros((C1S, Q2), np.float32)
        _d[_QV + G2 * _ki + _kj, _QV] = 1.0
        _D2_NP.append(_d)

_WF_SRC = np.array([640 * c + 112 * r + 4 * s
                    for c in range(C2) for r in range(5) for s in range(5)])
_WF_DST = np.array([128 * c + 26 * r + 2 * s
                    for c in range(C2) for r in range(5) for s in range(5)])


def _build_w1big(w1):
    w1r = w1.astype(_BF).reshape(C1, 9)
    acc = None
    for t in range(9):
        ki, kj = divmod(t, 3)
        d = jnp.eye(IMG * IMG, CH1S, k=-(IMG * ki + kj), dtype=_BF)
        term = (w1r[None, :, t, None] * d[:, None, :]).reshape(
            IMG * IMG, C1 * CH1S)
        acc = term if acc is None else acc + term
    return acc


def _build_w2big(w2):
    w2r = w2.astype(_BF).reshape(C2, C1, 3, 3)
    acc = None
    for t in range(9):
        ki, kj = divmod(t, 3)
        d = jnp.asarray(_D2_NP[t], dtype=_BF)
        term = (w2r[:, :, ki, kj].T[:, None, :, None]
                * d[None, :, None, :]).reshape(C1 * C1S, C2 * Q2)
        acc = term if acc is None else acc + term
    return acc


def _lenet_body(x_ref, w1b_ref, s_ref, b1_ref, w2b_ref, b2_ref,
                wf_ref, fb1_ref, w2f_ref, fb2_ref, w3f_ref, fb3_ref,
                out_ref):
    x = x_ref[...].astype(_BF)
    y1 = jnp.dot(x, w1b_ref[...], preferred_element_type=_F32)

    mcs = []
    sel = s_ref[...]
    for c in range(C1):
        b = c * CH1S
        m = jnp.maximum(
            jnp.maximum(y1[:, b:b + PW], y1[:, b + 1:b + 1 + PW]),
            jnp.maximum(y1[:, b + IMG:b + IMG + PW],
                        y1[:, b + IMG + 1:b + IMG + 1 + PW]))
        mcs.append(jnp.dot(m.astype(_BF), sel, preferred_element_type=_F32))
    m1c = jnp.concatenate(mcs, axis=1)
    a1 = jnp.maximum(m1c + b1_ref[...], 0.0).astype(_BF)

    z2 = jnp.dot(a1, w2b_ref[...], preferred_element_type=_F32)
    a2 = jnp.maximum(z2 + b2_ref[...], 0.0)

    m2s = []
    for o in range(C2):
        b = o * Q2
        m = jnp.maximum(
            jnp.maximum(a2[:, b:b + M2S], a2[:, b + 1:b + 1 + M2S]),
            jnp.maximum(a2[:, b + G2:b + G2 + M2S],
                        a2[:, b + G2 + 1:b + G2 + 1 + M2S]))
        m2s.append(m)
    m2 = jnp.concatenate(m2s, axis=1).astype(_BF)

    h = jnp.dot(m2, wf_ref[...], preferred_element_type=_F32)
    h = jnp.maximum(h + fb1_ref[...], 0.0)
    h = jnp.dot(h, w2f_ref[...], preferred_element_type=_F32)
    h = jnp.maximum(h + fb2_ref[...], 0.0)
    z = jnp.dot(h, w3f_ref[...], preferred_element_type=_F32) + fb3_ref[...]
    zmax = jnp.max(z, axis=-1, keepdims=True)
    lse = jnp.log(jnp.sum(jnp.exp(z - zmax), axis=-1, keepdims=True)) + zmax
    out_ref[...] = z - lse


def kernel(x, w1, b1, w2, b2, wf, fc1_b, fc2_wT, fc2_b, fc3_wT, fc3_b,
           *, block_n=256):
    n = x.shape[0]
    x_flat = x.reshape(n, -1)

    npad = ((n + block_n - 1) // block_n) * block_n
    if npad != n:
        x_flat = jnp.pad(x_flat, ((0, npad - n), (0, 0)))

    w1big = _build_w1big(w1)
    w2big = _build_w2big(w2)
    smat = jnp.asarray(_S_NP, dtype=_BF)
    b1big = jnp.repeat(b1, C1S).reshape(1, -1)
    b2big = jnp.repeat(b2, Q2).reshape(1, -1)
    wf2 = (jnp.zeros((C2 * M2S, FC1), _BF)
           .at[_WF_DST].set(wf[_WF_SRC].astype(_BF)))

    def resident(shape):
        return pl.BlockSpec(shape, lambda b: tuple(0 for _ in shape))

    out = pl.pallas_call(
        _lenet_body,
        out_shape=jax.ShapeDtypeStruct((npad, FC3), _F32),
        grid=(npad // block_n,),
        in_specs=[
            pl.BlockSpec((block_n, IMG * IMG), lambda b: (b, 0)),
            resident((IMG * IMG, C1 * CH1S)),
            resident((PW, C1S)),
            resident((1, C1 * C1S)),
            resident((C1 * C1S, C2 * Q2)),
            resident((1, C2 * Q2)),
            resident((C2 * M2S, FC1)),
            resident((1, FC1)),
            resident((FC1, FC2)), resident((1, FC2)),
            resident((FC2, FC3)), resident((1, FC3)),
        ],
        out_specs=pl.BlockSpec((block_n, FC3), lambda b: (b, 0)),
        compiler_params=pltpu.CompilerParams(
            dimension_semantics=("parallel",),
            vmem_limit_bytes=60 << 20,
        ),
    )(x_flat, w1big, smat, b1big, w2big, b2big, wf2,
      fc1_b, fc2_wT, fc2_b, fc3_wT, fc3_b)
    return out[:n]
